# Initial kernel scaffold; baseline (speedup 1.0000x reference)
import jax
import jax.numpy as jnp
from jax.experimental import pallas as pl


def kernel(x, edge_index, batch, idx_a, idx_b, W1, b1, W2, b2, Wfc, bfc):
    raise NotImplementedError("write your pallas kernel here")


# speedup vs baseline: 11.3669x; 11.3669x over previous
import functools

import jax
import jax.numpy as jnp
from jax import lax
from jax.experimental import pallas as pl
from jax.experimental.pallas import tpu as pltpu
from jax.experimental.pallas import tpu_sc as plsc

N = 10000
E = 320000
F_IN = 128
H = 64
G = 512
P = 1024

NC = 2
NS = 16
CH = 128
CPW = 80
E_PAD = NC * NS * CPW * CH
N_PAD = 10240
RPS = N_PAD // NS
DUMMY = N

BM = 256
GRID = N_PAD // BM

@functools.lru_cache(maxsize=None)
def _mesh():
    return plsc.VectorSubcoreMesh(
        core_axis_name="c", subcore_axis_name="s", num_cores=NC, num_subcores=NS
    )


def _fill(buf, value, ncols):
    v = jnp.full((16,), value, jnp.float32)

    def body(i, carry):
        for j in range(ncols):
            buf[i, pl.ds(j * 16, 16)] = v
        return carry

    lax.fori_loop(0, CH, body, 0)


@functools.lru_cache(maxsize=None)
def _build_sc_degree():
    return pl.kernel(
        _sc_degree_body,
        out_type=jax.ShapeDtypeStruct((NC, N_PAD, 16), jnp.float32),
        mesh=_mesh(),
        scratch_types=[
            pltpu.VMEM((CH,), jnp.int32),
            pltpu.VMEM((CH, 16), jnp.float32),
            pltpu.VMEM_SHARED((N_PAD, 16), jnp.float32),
        ],
        compiler_params=pltpu.CompilerParams(use_tc_tiling_on_sc=False),
    )


def _sc_degree(dst):
    return _build_sc_degree()(dst)


def _sc_degree_body(dst_hbm, out_hbm, didx_v, ones_v, deg_sh):
    c = lax.axis_index("c")
    s = lax.axis_index("s")
    wid = c * NS + s

    _fill(ones_v, 0.0, 1)

    def zbody(j, carry):
        pltpu.sync_copy(ones_v, deg_sh.at[pl.ds(s * RPS + j * CH, CH)])
        return carry

    lax.fori_loop(0, RPS // CH, zbody, 0)
    plsc.subcore_barrier()

    _fill(ones_v, 1.0, 1)

    def body(j, carry):
        off = (wid * CPW + j) * CH
        pltpu.sync_copy(dst_hbm.at[pl.ds(off, CH)], didx_v)
        pltpu.sync_copy(ones_v, deg_sh.at[didx_v], add=True)
        return carry

    lax.fori_loop(0, CPW, body, 0)
    plsc.subcore_barrier()
    pltpu.sync_copy(
        deg_sh.at[pl.ds(s * RPS, RPS)], out_hbm.at[c, pl.ds(s * RPS, RPS)]
    )


@functools.lru_cache(maxsize=None)
def _build_sc_edge_scatter():
    return pl.kernel(
        _sc_edge_scatter_body,
        out_type=jax.ShapeDtypeStruct((NC, N_PAD, H), jnp.float32),
        mesh=_mesh(),
        scratch_types=[
            pltpu.VMEM((CH,), jnp.int32),
            pltpu.VMEM((CH,), jnp.int32),
            pltpu.VMEM((CH, H), jnp.float32),
            pltpu.VMEM_SHARED((N_PAD, H), jnp.float32),
            pltpu.SemaphoreType.DMA,
        ],
        compiler_params=pltpu.CompilerParams(use_tc_tiling_on_sc=False),
    )


def _sc_edge_scatter(u, src, dst):
    return _build_sc_edge_scatter()(u, src, dst)


def _sc_edge_scatter_body(u_hbm, src_hbm, dst_hbm, out_hbm, sidx_v, didx_v, rows_v, acc_sh, sem):
    c = lax.axis_index("c")
    s = lax.axis_index("s")
    wid = c * NS + s

    _fill(rows_v, 0.0, H // 16)

    def zbody(j, carry):
        pltpu.sync_copy(rows_v, acc_sh.at[pl.ds(s * RPS + j * CH, CH)])
        return carry

    lax.fori_loop(0, RPS // CH, zbody, 0)
    plsc.subcore_barrier()

    def body(j, carry):
        off = (wid * CPW + j) * CH
        pltpu.sync_copy(src_hbm.at[pl.ds(off, CH)], sidx_v)
        pltpu.sync_copy(dst_hbm.at[pl.ds(off, CH)], didx_v)
        pltpu.async_copy(u_hbm.at[sidx_v], rows_v, sem).wait()
        pltpu.sync_copy(rows_v, acc_sh.at[didx_v], add=True)
        return carry

    lax.fori_loop(0, CPW, body, 0)
    plsc.subcore_barrier()
    pltpu.sync_copy(
        acc_sh.at[pl.ds(s * RPS, RPS)], out_hbm.at[c, pl.ds(s * RPS, RPS)]
    )


def _mm_body(x_ref, w_ref, o_ref):
    o_ref[...] = jnp.dot(
        x_ref[...].astype(jnp.bfloat16), w_ref[...].astype(jnp.bfloat16),
        preferred_element_type=jnp.float32,
    )


def _mm(x_pad, W1):
    return pl.pallas_call(
        _mm_body,
        grid=(GRID,),
        in_specs=[
            pl.BlockSpec((BM, F_IN), lambda i: (i, 0)),
            pl.BlockSpec((F_IN, H), lambda i: (0, 0)),
        ],
        out_specs=pl.BlockSpec((BM, H), lambda i: (i, 0)),
        out_shape=jax.ShapeDtypeStruct((N_PAD, H), jnp.float32),
    )(x_pad, W1)


def _scale_body(t_ref, d0_ref, d1_ref, u_ref, s_ref):
    i = pl.program_id(0)
    deg = d0_ref[...][:, 0:1] + d1_ref[...][:, 0:1]
    row = lax.broadcasted_iota(jnp.int32, (BM, 1), 0) + i * BM
    sc = jnp.where(row < N, 1.0 / jnp.sqrt(deg + 1.0), 0.0)
    s64 = jnp.broadcast_to(sc, (BM, H))
    s_ref[...] = s64
    u_ref[...] = t_ref[...] * s64


def _scale(t1, d0, d1):
    return pl.pallas_call(
        _scale_body,
        grid=(GRID,),
        in_specs=[
            pl.BlockSpec((BM, H), lambda i: (i, 0)),
            pl.BlockSpec((BM, 16), lambda i: (i, 0)),
            pl.BlockSpec((BM, 16), lambda i: (i, 0)),
        ],
        out_specs=[
            pl.BlockSpec((BM, H), lambda i: (i, 0)),
            pl.BlockSpec((BM, H), lambda i: (i, 0)),
        ],
        out_shape=[
            jax.ShapeDtypeStruct((N_PAD, H), jnp.float32),
            jax.ShapeDtypeStruct((N_PAD, H), jnp.float32),
        ],
    )(t1, d0, d1)


def _layer2_body(a0_ref, a1_ref, u1_ref, s_ref, w_ref, b_ref, o_ref):
    s = s_ref[...]
    h = jnp.maximum(s * (a0_ref[...] + a1_ref[...] + u1_ref[...]) + b_ref[...], 0.0)
    o_ref[...] = jnp.dot(
        h.astype(jnp.bfloat16), w_ref[...].astype(jnp.bfloat16),
        preferred_element_type=jnp.float32,
    ) * s


def _layer2(a0, a1, u1, s64, W2, b1_row):
    return pl.pallas_call(
        _layer2_body,
        grid=(GRID,),
        in_specs=[
            pl.BlockSpec((BM, H), lambda i: (i, 0)),
            pl.BlockSpec((BM, H), lambda i: (i, 0)),
            pl.BlockSpec((BM, H), lambda i: (i, 0)),
            pl.BlockSpec((BM, H), lambda i: (i, 0)),
            pl.BlockSpec((H, H), lambda i: (0, 0)),
            pl.BlockSpec((1, H), lambda i: (0, 0)),
        ],
        out_specs=pl.BlockSpec((BM, H), lambda i: (i, 0)),
        out_shape=jax.ShapeDtypeStruct((N_PAD, H), jnp.float32),
    )(a0, a1, u1, s64, W2, b1_row)


def _final_body(
    a0_ref, a1_ref, u2_ref, s_ref, b_ref, wfc_ref, bfc_ref, batch_ref,
    ia_ref, ib_ref, util_ref, pairs_ref, sum_sc, cnt_sc
):
    i = pl.program_id(0)

    @pl.when(i == 0)
    def _():
        sum_sc[...] = jnp.zeros_like(sum_sc)
        cnt_sc[...] = jnp.zeros_like(cnt_sc)

    s = s_ref[...]
    h = jnp.maximum(s * (a0_ref[...] + a1_ref[...] + u2_ref[...]) + b_ref[...], 0.0)
    z = jnp.dot(
        h.astype(jnp.bfloat16), wfc_ref[...].astype(jnp.bfloat16),
        preferred_element_type=jnp.float32,
    ) + bfc_ref[...]
    b = batch_ref[...]
    oh = (b == lax.broadcasted_iota(jnp.int32, (BM, G), 1)).astype(jnp.float32)
    dn = (((0,), (0,)), ((), ()))
    sum_sc[...] += lax.dot_general(oh, z, dn, preferred_element_type=jnp.float32, precision=lax.Precision.HIGHEST)
    cnt_sc[...] += lax.dot_general(
        oh, jnp.ones((BM, 1), jnp.float32), dn, preferred_element_type=jnp.float32,
        precision=lax.Precision.HIGHEST
    )

    @pl.when(i == GRID - 1)
    def _():
        util = sum_sc[...] / jnp.clip(cnt_sc[...], 1.0, None)
        util_ref[...] = util
        iot = lax.broadcasted_iota(jnp.int32, (P, G), 1)
        d = (ib_ref[...] == iot).astype(jnp.float32) - (
            ia_ref[...] == iot
        ).astype(jnp.float32)
        pairs_ref[...] = lax.dot_general(
            d, util, (((1,), (0,)), ((), ())), preferred_element_type=jnp.float32,
            precision=lax.Precision.HIGHEST
        )


def _final(a0, a1, u2, s64, b2_row, Wfc, bfc_row, batch_pad, ia, ib):
    return pl.pallas_call(
        _final_body,
        grid=(GRID,),
        in_specs=[
            pl.BlockSpec((BM, H), lambda i: (i, 0)),
            pl.BlockSpec((BM, H), lambda i: (i, 0)),
            pl.BlockSpec((BM, H), lambda i: (i, 0)),
            pl.BlockSpec((BM, H), lambda i: (i, 0)),
            pl.BlockSpec((1, H), lambda i: (0, 0)),
            pl.BlockSpec((H, 1), lambda i: (0, 0)),
            pl.BlockSpec((1, 1), lambda i: (0, 0)),
            pl.BlockSpec((BM, 1), lambda i: (i, 0)),
            pl.BlockSpec((P, 1), lambda i: (0, 0)),
            pl.BlockSpec((P, 1), lambda i: (0, 0)),
        ],
        out_specs=[
            pl.BlockSpec((G, 1), lambda i: (0, 0)),
            pl.BlockSpec((P, 1), lambda i: (0, 0)),
        ],
        out_shape=[
            jax.ShapeDtypeStruct((G, 1), jnp.float32),
            jax.ShapeDtypeStruct((P, 1), jnp.float32),
        ],
        scratch_shapes=[
            pltpu.VMEM((G, 1), jnp.float32),
            pltpu.VMEM((G, 1), jnp.float32),
        ],
    )(a0, a1, u2, s64, b2_row, Wfc, bfc_row, batch_pad, ia, ib)


def kernel(x, edge_index, batch, idx_a, idx_b, W1, b1, W2, b2, Wfc, bfc):
    x_pad = jnp.zeros((N_PAD, F_IN), jnp.float32).at[:N, :].set(x)
    pad_idx = jnp.full((E_PAD - E,), DUMMY, jnp.int32)
    src = jnp.concatenate([edge_index[0], pad_idx])
    dst = jnp.concatenate([edge_index[1], pad_idx])
    batch_pad = jnp.concatenate(
        [batch, jnp.full((N_PAD - N,), G, jnp.int32)]
    ).reshape(N_PAD, 1)
    ia = idx_a.reshape(P, 1)
    ib = idx_b.reshape(P, 1)

    deg2 = _sc_degree(dst)
    t1 = _mm(x_pad, W1)
    u1, s64 = _scale(t1, deg2[0], deg2[1])
    acc1 = _sc_edge_scatter(u1, src, dst)
    u2 = _layer2(acc1[0], acc1[1], u1, s64, W2, b1.reshape(1, H))
    acc2 = _sc_edge_scatter(u2, src, dst)
    util, pairs = _final(
        acc2[0], acc2[1], u2, s64, b2.reshape(1, H), Wfc, bfc.reshape(1, 1),
        batch_pad, ia, ib
    )
    return pairs.reshape(P), util


# speedup vs baseline: 14.5606x; 1.2810x over previous
import functools

import jax
import jax.numpy as jnp
from jax import lax
from jax.experimental import pallas as pl
from jax.experimental.pallas import tpu as pltpu
from jax.experimental.pallas import tpu_sc as plsc

N = 10000
E = 320000
F_IN = 128
H = 64
G = 512
P = 1024

NC = 2
NS = 16
CH = 128
CPW = 80
E_PAD = NC * NS * CPW * CH
N_PAD = 10240
RPS = N_PAD // NS
DUMMY = N

BM = 256
GRID = N_PAD // BM

@functools.lru_cache(maxsize=None)
def _mesh():
    return plsc.VectorSubcoreMesh(
        core_axis_name="c", subcore_axis_name="s", num_cores=NC, num_subcores=NS
    )


def _fill(buf, value, ncols):
    v = jnp.full((16,), value, jnp.float32)

    def body(i, carry):
        for j in range(ncols):
            buf[i, pl.ds(j * 16, 16)] = v
        return carry

    lax.fori_loop(0, CH, body, 0)


@functools.lru_cache(maxsize=None)
def _build_sc_degree():
    return pl.kernel(
        _sc_degree_body,
        out_type=jax.ShapeDtypeStruct((NC, N_PAD, 16), jnp.float32),
        mesh=_mesh(),
        scratch_types=[
            pltpu.VMEM((CH,), jnp.int32),
            pltpu.VMEM((CH, 16), jnp.float32),
            pltpu.VMEM_SHARED((N_PAD, 16), jnp.float32),
        ],
        compiler_params=pltpu.CompilerParams(use_tc_tiling_on_sc=False),
    )


def _sc_degree(dst):
    return _build_sc_degree()(dst)


def _sc_degree_body(dst_hbm, out_hbm, didx_v, ones_v, deg_sh):
    c = lax.axis_index("c")
    s = lax.axis_index("s")
    wid = c * NS + s

    _fill(ones_v, 0.0, 1)

    def zbody(j, carry):
        pltpu.sync_copy(ones_v, deg_sh.at[pl.ds(s * RPS + j * CH, CH)])
        return carry

    lax.fori_loop(0, RPS // CH, zbody, 0)
    plsc.subcore_barrier()

    _fill(ones_v, 1.0, 1)

    def body(j, carry):
        off = (wid * CPW + j) * CH
        pltpu.sync_copy(dst_hbm.at[pl.ds(off, CH)], didx_v)
        pltpu.sync_copy(ones_v, deg_sh.at[didx_v], add=True)
        return carry

    lax.fori_loop(0, CPW, body, 0)
    plsc.subcore_barrier()
    pltpu.sync_copy(
        deg_sh.at[pl.ds(s * RPS, RPS)], out_hbm.at[c, pl.ds(s * RPS, RPS)]
    )


@functools.lru_cache(maxsize=None)
def _build_sc_edge_scatter():
    return pl.kernel(
        _sc_edge_scatter_body,
        out_type=jax.ShapeDtypeStruct((NC, N_PAD, H), jnp.float32),
        mesh=_mesh(),
        scratch_types=[
            pltpu.VMEM((2, CH), jnp.int32),
            pltpu.VMEM((2, CH), jnp.int32),
            pltpu.VMEM((2, CH, H), jnp.float32),
            pltpu.VMEM_SHARED((N_PAD, H), jnp.float32),
            pltpu.SemaphoreType.DMA,
            pltpu.SemaphoreType.DMA,
        ],
        compiler_params=pltpu.CompilerParams(use_tc_tiling_on_sc=False),
    )


def _sc_edge_scatter(u, src, dst):
    return _build_sc_edge_scatter()(u, src, dst)


def _sc_edge_scatter_body(u_hbm, src_hbm, dst_hbm, out_hbm, sidx_v, didx_v, rows_v, acc_sh, sem0, sem1):
    c = lax.axis_index("c")
    s = lax.axis_index("s")
    wid = c * NS + s
    sems = (sem0, sem1)

    _fill(rows_v.at[0], 0.0, H // 16)

    def zbody(j, carry):
        pltpu.sync_copy(rows_v.at[0], acc_sh.at[pl.ds(s * RPS + j * CH, CH)])
        return carry

    lax.fori_loop(0, RPS // CH, zbody, 0)
    plsc.subcore_barrier()

    base = wid * CPW
    pltpu.sync_copy(src_hbm.at[pl.ds(base * CH, CH)], sidx_v.at[0])
    pltpu.sync_copy(dst_hbm.at[pl.ds(base * CH, CH)], didx_v.at[0])
    pltpu.async_copy(u_hbm.at[sidx_v.at[0]], rows_v.at[0], sems[0])

    def body(i, carry):
        for b in range(2):
            j = 2 * i + b
            nb = 1 - b
            offn = (base + j + 1) * CH
            pltpu.sync_copy(src_hbm.at[pl.ds(offn, CH)], sidx_v.at[nb])
            pltpu.sync_copy(dst_hbm.at[pl.ds(offn, CH)], didx_v.at[nb])
            pltpu.async_copy(u_hbm.at[sidx_v.at[nb]], rows_v.at[nb], sems[nb])
            pltpu.make_async_copy(u_hbm.at[sidx_v.at[b]], rows_v.at[b], sems[b]).wait()
            pltpu.sync_copy(rows_v.at[b], acc_sh.at[didx_v.at[b]], add=True)
        return carry

    lax.fori_loop(0, CPW // 2, body, 0)
    pltpu.make_async_copy(u_hbm.at[sidx_v.at[0]], rows_v.at[0], sems[0]).wait()

    plsc.subcore_barrier()
    pltpu.sync_copy(
        acc_sh.at[pl.ds(s * RPS, RPS)], out_hbm.at[c, pl.ds(s * RPS, RPS)]
    )


def _mm_body(x_ref, w_ref, o_ref):
    o_ref[...] = jnp.dot(
        x_ref[...].astype(jnp.bfloat16), w_ref[...].astype(jnp.bfloat16),
        preferred_element_type=jnp.float32,
    )


def _mm(x_pad, W1):
    return pl.pallas_call(
        _mm_body,
        grid=(GRID,),
        in_specs=[
            pl.BlockSpec((BM, F_IN), lambda i: (i, 0)),
            pl.BlockSpec((F_IN, H), lambda i: (0, 0)),
        ],
        out_specs=pl.BlockSpec((BM, H), lambda i: (i, 0)),
        out_shape=jax.ShapeDtypeStruct((N_PAD, H), jnp.float32),
    )(x_pad, W1)


def _scale_body(t_ref, d0_ref, d1_ref, u_ref, s_ref):
    i = pl.program_id(0)
    deg = d0_ref[...][:, 0:1] + d1_ref[...][:, 0:1]
    row = lax.broadcasted_iota(jnp.int32, (BM, 1), 0) + i * BM
    sc = jnp.where(row < N, 1.0 / jnp.sqrt(deg + 1.0), 0.0)
    s64 = jnp.broadcast_to(sc, (BM, H))
    s_ref[...] = s64
    u_ref[...] = t_ref[...] * s64


def _scale(t1, d0, d1):
    return pl.pallas_call(
        _scale_body,
        grid=(GRID,),
        in_specs=[
            pl.BlockSpec((BM, H), lambda i: (i, 0)),
            pl.BlockSpec((BM, 16), lambda i: (i, 0)),
            pl.BlockSpec((BM, 16), lambda i: (i, 0)),
        ],
        out_specs=[
            pl.BlockSpec((BM, H), lambda i: (i, 0)),
            pl.BlockSpec((BM, H), lambda i: (i, 0)),
        ],
        out_shape=[
            jax.ShapeDtypeStruct((N_PAD, H), jnp.float32),
            jax.ShapeDtypeStruct((N_PAD, H), jnp.float32),
        ],
    )(t1, d0, d1)


def _layer2_body(a0_ref, a1_ref, u1_ref, s_ref, w_ref, b_ref, o_ref):
    s = s_ref[...]
    h = jnp.maximum(s * (a0_ref[...] + a1_ref[...] + u1_ref[...]) + b_ref[...], 0.0)
    o_ref[...] = jnp.dot(
        h.astype(jnp.bfloat16), w_ref[...].astype(jnp.bfloat16),
        preferred_element_type=jnp.float32,
    ) * s


def _layer2(a0, a1, u1, s64, W2, b1_row):
    return pl.pallas_call(
        _layer2_body,
        grid=(GRID,),
        in_specs=[
            pl.BlockSpec((BM, H), lambda i: (i, 0)),
            pl.BlockSpec((BM, H), lambda i: (i, 0)),
            pl.BlockSpec((BM, H), lambda i: (i, 0)),
            pl.BlockSpec((BM, H), lambda i: (i, 0)),
            pl.BlockSpec((H, H), lambda i: (0, 0)),
            pl.BlockSpec((1, H), lambda i: (0, 0)),
        ],
        out_specs=pl.BlockSpec((BM, H), lambda i: (i, 0)),
        out_shape=jax.ShapeDtypeStruct((N_PAD, H), jnp.float32),
    )(a0, a1, u1, s64, W2, b1_row)


def _final_body(
    a0_ref, a1_ref, u2_ref, s_ref, b_ref, wfc_ref, bfc_ref, batch_ref,
    ia_ref, ib_ref, util_ref, pairs_ref, sum_sc, cnt_sc
):
    i = pl.program_id(0)

    @pl.when(i == 0)
    def _():
        sum_sc[...] = jnp.zeros_like(sum_sc)
        cnt_sc[...] = jnp.zeros_like(cnt_sc)

    s = s_ref[...]
    h = jnp.maximum(s * (a0_ref[...] + a1_ref[...] + u2_ref[...]) + b_ref[...], 0.0)
    z = jnp.dot(
        h.astype(jnp.bfloat16), wfc_ref[...].astype(jnp.bfloat16),
        preferred_element_type=jnp.float32,
    ) + bfc_ref[...]
    b = batch_ref[...]
    oh = (b == lax.broadcasted_iota(jnp.int32, (BM, G), 1)).astype(jnp.float32)
    dn = (((0,), (0,)), ((), ()))
    sum_sc[...] += lax.dot_general(oh, z, dn, preferred_element_type=jnp.float32, precision=lax.Precision.HIGHEST)
    cnt_sc[...] += lax.dot_general(
        oh, jnp.ones((BM, 1), jnp.float32), dn, preferred_element_type=jnp.float32,
        precision=lax.Precision.HIGHEST
    )

    @pl.when(i == GRID - 1)
    def _():
        util = sum_sc[...] / jnp.clip(cnt_sc[...], 1.0, None)
        util_ref[...] = util
        iot = lax.broadcasted_iota(jnp.int32, (P, G), 1)
        d = (ib_ref[...] == iot).astype(jnp.float32) - (
            ia_ref[...] == iot
        ).astype(jnp.float32)
        pairs_ref[...] = lax.dot_general(
            d, util, (((1,), (0,)), ((), ())), preferred_element_type=jnp.float32,
            precision=lax.Precision.HIGHEST
        )


def _final(a0, a1, u2, s64, b2_row, Wfc, bfc_row, batch_pad, ia, ib):
    return pl.pallas_call(
        _final_body,
        grid=(GRID,),
        in_specs=[
            pl.BlockSpec((BM, H), lambda i: (i, 0)),
            pl.BlockSpec((BM, H), lambda i: (i, 0)),
            pl.BlockSpec((BM, H), lambda i: (i, 0)),
            pl.BlockSpec((BM, H), lambda i: (i, 0)),
            pl.BlockSpec((1, H), lambda i: (0, 0)),
            pl.BlockSpec((H, 1), lambda i: (0, 0)),
            pl.BlockSpec((1, 1), lambda i: (0, 0)),
            pl.BlockSpec((BM, 1), lambda i: (i, 0)),
            pl.BlockSpec((P, 1), lambda i: (0, 0)),
            pl.BlockSpec((P, 1), lambda i: (0, 0)),
        ],
        out_specs=[
            pl.BlockSpec((G, 1), lambda i: (0, 0)),
            pl.BlockSpec((P, 1), lambda i: (0, 0)),
        ],
        out_shape=[
            jax.ShapeDtypeStruct((G, 1), jnp.float32),
            jax.ShapeDtypeStruct((P, 1), jnp.float32),
        ],
        scratch_shapes=[
            pltpu.VMEM((G, 1), jnp.float32),
            pltpu.VMEM((G, 1), jnp.float32),
        ],
    )(a0, a1, u2, s64, b2_row, Wfc, bfc_row, batch_pad, ia, ib)


def kernel(x, edge_index, batch, idx_a, idx_b, W1, b1, W2, b2, Wfc, bfc):
    x_pad = jnp.zeros((N_PAD, F_IN), jnp.float32).at[:N, :].set(x)
    pad_idx = jnp.full((E_PAD + CH - E,), DUMMY, jnp.int32)
    src = jnp.concatenate([edge_index[0], pad_idx])
    dst = jnp.concatenate([edge_index[1], pad_idx])
    batch_pad = jnp.concatenate(
        [batch, jnp.full((N_PAD - N,), G, jnp.int32)]
    ).reshape(N_PAD, 1)
    ia = idx_a.reshape(P, 1)
    ib = idx_b.reshape(P, 1)

    deg2 = _sc_degree(dst)
    t1 = _mm(x_pad, W1)
    u1, s64 = _scale(t1, deg2[0], deg2[1])
    acc1 = _sc_edge_scatter(u1, src, dst)
    u2 = _layer2(acc1[0], acc1[1], u1, s64, W2, b1.reshape(1, H))
    acc2 = _sc_edge_scatter(u2, src, dst)
    util, pairs = _final(
        acc2[0], acc2[1], u2, s64, b2.reshape(1, H), Wfc, bfc.reshape(1, 1),
        batch_pad, ia, ib
    )
    return pairs.reshape(P), util


# speedup vs baseline: 15.4983x; 1.0644x over previous
import functools

import jax
import jax.numpy as jnp
from jax import lax
from jax.experimental import pallas as pl
from jax.experimental.pallas import tpu as pltpu
from jax.experimental.pallas import tpu_sc as plsc

N = 10000
E = 320000
F_IN = 128
H = 64
G = 512
P = 1024

NC = 2
NS = 16
CH = 128
CPW = 80
E_PAD = NC * NS * CPW * CH
N_PAD = 10240
RPS = N_PAD // NS
DUMMY = N

BM = 256
GRID = N_PAD // BM

@functools.lru_cache(maxsize=None)
def _mesh():
    return plsc.VectorSubcoreMesh(
        core_axis_name="c", subcore_axis_name="s", num_cores=NC, num_subcores=NS
    )


def _fill(buf, value, ncols):
    v = jnp.full((16,), value, jnp.float32)

    def body(i, carry):
        for j in range(ncols):
            buf[i, pl.ds(j * 16, 16)] = v
        return carry

    lax.fori_loop(0, CH, body, 0)


@functools.lru_cache(maxsize=None)
def _build_sc_degree():
    return pl.kernel(
        _sc_degree_body,
        out_type=jax.ShapeDtypeStruct((NC, N_PAD, 16), jnp.float32),
        mesh=_mesh(),
        scratch_types=[
            pltpu.VMEM((CPW, CH), jnp.int32),
            pltpu.VMEM((CH, 16), jnp.float32),
            pltpu.VMEM_SHARED((N_PAD, 16), jnp.float32),
        ],
        compiler_params=pltpu.CompilerParams(use_tc_tiling_on_sc=False),
    )


def _sc_degree(dst):
    return _build_sc_degree()(dst)


def _sc_degree_body(dst_hbm, out_hbm, didx_v, ones_v, deg_sh):
    c = lax.axis_index("c")
    s = lax.axis_index("s")
    wid = c * NS + s

    _fill(ones_v, 0.0, 1)

    def zbody(j, carry):
        pltpu.sync_copy(ones_v, deg_sh.at[pl.ds(s * RPS + j * CH, CH)])
        return carry

    lax.fori_loop(0, RPS // CH, zbody, 0)
    plsc.subcore_barrier()

    _fill(ones_v, 1.0, 1)
    pltpu.sync_copy(dst_hbm.at[pl.ds(wid * CPW, CPW)], didx_v)

    def body(j, carry):
        pltpu.sync_copy(ones_v, deg_sh.at[didx_v.at[j]], add=True)
        return carry

    lax.fori_loop(0, CPW, body, 0)
    plsc.subcore_barrier()
    pltpu.sync_copy(
        deg_sh.at[pl.ds(s * RPS, RPS)], out_hbm.at[c, pl.ds(s * RPS, RPS)]
    )


@functools.lru_cache(maxsize=None)
def _build_sc_edge_scatter():
    return pl.kernel(
        _sc_edge_scatter_body,
        out_type=jax.ShapeDtypeStruct((NC, N_PAD, H), jnp.float32),
        mesh=_mesh(),
        scratch_types=[
            pltpu.VMEM((CPW + 1, CH), jnp.int32),
            pltpu.VMEM((CPW, CH), jnp.int32),
            pltpu.VMEM((2, CH, H), jnp.float32),
            pltpu.VMEM_SHARED((N_PAD, H), jnp.float32),
            pltpu.SemaphoreType.DMA,
            pltpu.SemaphoreType.DMA,
        ],
        compiler_params=pltpu.CompilerParams(use_tc_tiling_on_sc=False),
    )


def _sc_edge_scatter(u, src, dst):
    return _build_sc_edge_scatter()(u, src, dst)


def _sc_edge_scatter_body(u_hbm, src_hbm, dst_hbm, out_hbm, sidx_v, didx_v, rows_v, acc_sh, sem0, sem1):
    c = lax.axis_index("c")
    s = lax.axis_index("s")
    wid = c * NS + s
    sems = (sem0, sem1)

    _fill(rows_v.at[0], 0.0, H // 16)

    def zbody(j, carry):
        pltpu.sync_copy(rows_v.at[0], acc_sh.at[pl.ds(s * RPS + j * CH, CH)])
        return carry

    lax.fori_loop(0, RPS // CH, zbody, 0)
    plsc.subcore_barrier()

    base = wid * CPW
    pltpu.sync_copy(src_hbm.at[pl.ds(base, CPW + 1)], sidx_v)
    pltpu.sync_copy(dst_hbm.at[pl.ds(base, CPW)], didx_v)
    pltpu.async_copy(u_hbm.at[sidx_v.at[0]], rows_v.at[0], sems[0])

    def body(i, carry):
        for b in range(2):
            j = 2 * i + b
            nb = 1 - b
            pltpu.async_copy(u_hbm.at[sidx_v.at[j + 1]], rows_v.at[nb], sems[nb])
            pltpu.make_async_copy(u_hbm.at[sidx_v.at[j]], rows_v.at[b], sems[b]).wait()
            pltpu.sync_copy(rows_v.at[b], acc_sh.at[didx_v.at[j]], add=True)
        return carry

    lax.fori_loop(0, CPW // 2, body, 0)
    pltpu.make_async_copy(u_hbm.at[sidx_v.at[CPW]], rows_v.at[0], sems[0]).wait()

    plsc.subcore_barrier()
    pltpu.sync_copy(
        acc_sh.at[pl.ds(s * RPS, RPS)], out_hbm.at[c, pl.ds(s * RPS, RPS)]
    )


def _mm_body(x_ref, w_ref, o_ref):
    o_ref[...] = jnp.dot(
        x_ref[...].astype(jnp.bfloat16), w_ref[...].astype(jnp.bfloat16),
        preferred_element_type=jnp.float32,
    )


def _mm(x_pad, W1):
    return pl.pallas_call(
        _mm_body,
        grid=(GRID,),
        in_specs=[
            pl.BlockSpec((BM, F_IN), lambda i: (i, 0)),
            pl.BlockSpec((F_IN, H), lambda i: (0, 0)),
        ],
        out_specs=pl.BlockSpec((BM, H), lambda i: (i, 0)),
        out_shape=jax.ShapeDtypeStruct((N_PAD, H), jnp.float32),
    )(x_pad, W1)


def _scale_body(t_ref, d0_ref, d1_ref, u_ref, s_ref):
    i = pl.program_id(0)
    deg = d0_ref[...][:, 0:1] + d1_ref[...][:, 0:1]
    row = lax.broadcasted_iota(jnp.int32, (BM, 1), 0) + i * BM
    sc = jnp.where(row < N, 1.0 / jnp.sqrt(deg + 1.0), 0.0)
    s64 = jnp.broadcast_to(sc, (BM, H))
    s_ref[...] = s64
    u_ref[...] = t_ref[...] * s64


def _scale(t1, d0, d1):
    return pl.pallas_call(
        _scale_body,
        grid=(GRID,),
        in_specs=[
            pl.BlockSpec((BM, H), lambda i: (i, 0)),
            pl.BlockSpec((BM, 16), lambda i: (i, 0)),
            pl.BlockSpec((BM, 16), lambda i: (i, 0)),
        ],
        out_specs=[
            pl.BlockSpec((BM, H), lambda i: (i, 0)),
            pl.BlockSpec((BM, H), lambda i: (i, 0)),
        ],
        out_shape=[
            jax.ShapeDtypeStruct((N_PAD, H), jnp.float32),
            jax.ShapeDtypeStruct((N_PAD, H), jnp.float32),
        ],
    )(t1, d0, d1)


def _layer2_body(a0_ref, a1_ref, u1_ref, s_ref, w_ref, b_ref, o_ref):
    s = s_ref[...]
    h = jnp.maximum(s * (a0_ref[...] + a1_ref[...] + u1_ref[...]) + b_ref[...], 0.0)
    o_ref[...] = jnp.dot(
        h.astype(jnp.bfloat16), w_ref[...].astype(jnp.bfloat16),
        preferred_element_type=jnp.float32,
    ) * s


def _layer2(a0, a1, u1, s64, W2, b1_row):
    return pl.pallas_call(
        _layer2_body,
        grid=(GRID,),
        in_specs=[
            pl.BlockSpec((BM, H), lambda i: (i, 0)),
            pl.BlockSpec((BM, H), lambda i: (i, 0)),
            pl.BlockSpec((BM, H), lambda i: (i, 0)),
            pl.BlockSpec((BM, H), lambda i: (i, 0)),
            pl.BlockSpec((H, H), lambda i: (0, 0)),
            pl.BlockSpec((1, H), lambda i: (0, 0)),
        ],
        out_specs=pl.BlockSpec((BM, H), lambda i: (i, 0)),
        out_shape=jax.ShapeDtypeStruct((N_PAD, H), jnp.float32),
    )(a0, a1, u1, s64, W2, b1_row)


def _final_body(
    a0_ref, a1_ref, u2_ref, s_ref, b_ref, wfc_ref, bfc_ref, batch_ref,
    ia_ref, ib_ref, util_ref, pairs_ref, sum_sc, cnt_sc
):
    i = pl.program_id(0)

    @pl.when(i == 0)
    def _():
        sum_sc[...] = jnp.zeros_like(sum_sc)
        cnt_sc[...] = jnp.zeros_like(cnt_sc)

    s = s_ref[...]
    h = jnp.maximum(s * (a0_ref[...] + a1_ref[...] + u2_ref[...]) + b_ref[...], 0.0)
    z = jnp.dot(
        h.astype(jnp.bfloat16), wfc_ref[...].astype(jnp.bfloat16),
        preferred_element_type=jnp.float32,
    ) + bfc_ref[...]
    b = batch_ref[...]
    oh = (b == lax.broadcasted_iota(jnp.int32, (BM, G), 1)).astype(jnp.float32)
    dn = (((0,), (0,)), ((), ()))
    sum_sc[...] += lax.dot_general(oh, z, dn, preferred_element_type=jnp.float32, precision=lax.Precision.HIGHEST)
    cnt_sc[...] += lax.dot_general(
        oh, jnp.ones((BM, 1), jnp.float32), dn, preferred_element_type=jnp.float32,
        precision=lax.Precision.HIGHEST
    )

    @pl.when(i == GRID - 1)
    def _():
        util = sum_sc[...] / jnp.clip(cnt_sc[...], 1.0, None)
        util_ref[...] = util
        iot = lax.broadcasted_iota(jnp.int32, (P, G), 1)
        d = (ib_ref[...] == iot).astype(jnp.float32) - (
            ia_ref[...] == iot
        ).astype(jnp.float32)
        pairs_ref[...] = lax.dot_general(
            d, util, (((1,), (0,)), ((), ())), preferred_element_type=jnp.float32,
            precision=lax.Precision.HIGHEST
        )


def _final(a0, a1, u2, s64, b2_row, Wfc, bfc_row, batch_pad, ia, ib):
    return pl.pallas_call(
        _final_body,
        grid=(GRID,),
        in_specs=[
            pl.BlockSpec((BM, H), lambda i: (i, 0)),
            pl.BlockSpec((BM, H), lambda i: (i, 0)),
            pl.BlockSpec((BM, H), lambda i: (i, 0)),
            pl.BlockSpec((BM, H), lambda i: (i, 0)),
            pl.BlockSpec((1, H), lambda i: (0, 0)),
            pl.BlockSpec((H, 1), lambda i: (0, 0)),
            pl.BlockSpec((1, 1), lambda i: (0, 0)),
            pl.BlockSpec((BM, 1), lambda i: (i, 0)),
            pl.BlockSpec((P, 1), lambda i: (0, 0)),
            pl.BlockSpec((P, 1), lambda i: (0, 0)),
        ],
        out_specs=[
            pl.BlockSpec((G, 1), lambda i: (0, 0)),
            pl.BlockSpec((P, 1), lambda i: (0, 0)),
        ],
        out_shape=[
            jax.ShapeDtypeStruct((G, 1), jnp.float32),
            jax.ShapeDtypeStruct((P, 1), jnp.float32),
        ],
        scratch_shapes=[
            pltpu.VMEM((G, 1), jnp.float32),
            pltpu.VMEM((G, 1), jnp.float32),
        ],
    )(a0, a1, u2, s64, b2_row, Wfc, bfc_row, batch_pad, ia, ib)


def kernel(x, edge_index, batch, idx_a, idx_b, W1, b1, W2, b2, Wfc, bfc):
    x_pad = jnp.zeros((N_PAD, F_IN), jnp.float32).at[:N, :].set(x)
    pad_idx = jnp.full((E_PAD + CH - E,), DUMMY, jnp.int32)
    src = jnp.concatenate([edge_index[0], pad_idx]).reshape(-1, CH)
    dst = jnp.concatenate([edge_index[1], pad_idx]).reshape(-1, CH)
    batch_pad = jnp.concatenate(
        [batch, jnp.full((N_PAD - N,), G, jnp.int32)]
    ).reshape(N_PAD, 1)
    ia = idx_a.reshape(P, 1)
    ib = idx_b.reshape(P, 1)

    deg2 = _sc_degree(dst)
    t1 = _mm(x_pad, W1)
    u1, s64 = _scale(t1, deg2[0], deg2[1])
    acc1 = _sc_edge_scatter(u1, src, dst)
    u2 = _layer2(acc1[0], acc1[1], u1, s64, W2, b1.reshape(1, H))
    acc2 = _sc_edge_scatter(u2, src, dst)
    util, pairs = _final(
        acc2[0], acc2[1], u2, s64, b2.reshape(1, H), Wfc, bfc.reshape(1, 1),
        batch_pad, ia, ib
    )
    return pairs.reshape(P), util


# speedup vs baseline: 25.4732x; 1.6436x over previous
import functools

import jax
import jax.numpy as jnp
from jax import lax
from jax.experimental import pallas as pl
from jax.experimental.pallas import tpu as pltpu
from jax.experimental.pallas import tpu_sc as plsc

N = 10000
E = 320000
F_IN = 128
H = 64
G = 512
P = 1024

NC = 2
NS = 16
CH = 128
CPW = 80
E_PAD = NC * NS * CPW * CH
N_PAD = 10240
RPS = N_PAD // NS
DUMMY = N

BM = 256
GRID = N_PAD // BM

@functools.lru_cache(maxsize=None)
def _mesh():
    return plsc.VectorSubcoreMesh(
        core_axis_name="c", subcore_axis_name="s", num_cores=NC, num_subcores=NS
    )


def _fill(buf, value, ncols):
    v = jnp.full((16,), value, jnp.float32)

    def body(i, carry):
        for j in range(ncols):
            buf[i, pl.ds(j * 16, 16)] = v
        return carry

    lax.fori_loop(0, CH, body, 0)


@functools.lru_cache(maxsize=None)
def _build_sc_degree():
    return pl.kernel(
        _sc_degree_body,
        out_type=jax.ShapeDtypeStruct((NC, N_PAD, 16), jnp.float32),
        mesh=_mesh(),
        scratch_types=[
            pltpu.VMEM((CPW, CH), jnp.int32),
            pltpu.VMEM((CH, 16), jnp.float32),
            pltpu.VMEM_SHARED((N_PAD, 16), jnp.float32),
        ],
        compiler_params=pltpu.CompilerParams(use_tc_tiling_on_sc=False),
    )


def _sc_degree(dst):
    return _build_sc_degree()(dst)


def _sc_degree_body(dst_hbm, out_hbm, didx_v, ones_v, deg_sh):
    c = lax.axis_index("c")
    s = lax.axis_index("s")
    wid = c * NS + s

    _fill(ones_v, 0.0, 1)

    def zbody(j, carry):
        pltpu.sync_copy(ones_v, deg_sh.at[pl.ds(s * RPS + j * CH, CH)])
        return carry

    lax.fori_loop(0, RPS // CH, zbody, 0)
    plsc.subcore_barrier()

    _fill(ones_v, 1.0, 1)
    pltpu.sync_copy(dst_hbm.at[pl.ds(wid * CPW, CPW)], didx_v)

    def body(j, carry):
        pltpu.sync_copy(ones_v, deg_sh.at[didx_v.at[j]], add=True)
        return carry

    lax.fori_loop(0, CPW, body, 0)
    plsc.subcore_barrier()
    pltpu.sync_copy(
        deg_sh.at[pl.ds(s * RPS, RPS)], out_hbm.at[c, pl.ds(s * RPS, RPS)]
    )


@functools.lru_cache(maxsize=None)
def _build_sc_edge_scatter():
    return pl.kernel(
        _sc_edge_scatter_body,
        out_type=jax.ShapeDtypeStruct((NC, N_PAD, H), jnp.float32),
        mesh=_mesh(),
        scratch_types=[
            pltpu.VMEM((CPW + 1, CH), jnp.int32),
            pltpu.VMEM((CPW, CH), jnp.int32),
            pltpu.VMEM((2, CH, H), jnp.float32),
            pltpu.VMEM_SHARED((N_PAD, H), jnp.float32),
            pltpu.VMEM_SHARED((N_PAD, H), jnp.float32),
            pltpu.SemaphoreType.DMA,
            pltpu.SemaphoreType.DMA,
        ],
        compiler_params=pltpu.CompilerParams(use_tc_tiling_on_sc=False),
    )


def _sc_edge_scatter(u, src, dst):
    return _build_sc_edge_scatter()(u, src, dst)


def _sc_edge_scatter_body(u_hbm, src_hbm, dst_hbm, out_hbm, sidx_v, didx_v, rows_v, acc_sh, u_sh, sem0, sem1):
    c = lax.axis_index("c")
    s = lax.axis_index("s")
    wid = c * NS + s
    sems = (sem0, sem1)

    pltpu.sync_copy(u_hbm.at[pl.ds(s * RPS, RPS)], u_sh.at[pl.ds(s * RPS, RPS)])

    _fill(rows_v.at[0], 0.0, H // 16)

    def zbody(j, carry):
        pltpu.sync_copy(rows_v.at[0], acc_sh.at[pl.ds(s * RPS + j * CH, CH)])
        return carry

    lax.fori_loop(0, RPS // CH, zbody, 0)
    plsc.subcore_barrier()

    base = wid * CPW
    pltpu.sync_copy(src_hbm.at[pl.ds(base, CPW + 1)], sidx_v)
    pltpu.sync_copy(dst_hbm.at[pl.ds(base, CPW)], didx_v)
    pltpu.async_copy(u_sh.at[sidx_v.at[0]], rows_v.at[0], sems[0])

    def body(i, carry):
        for b in range(2):
            j = 2 * i + b
            nb = 1 - b
            pltpu.async_copy(u_sh.at[sidx_v.at[j + 1]], rows_v.at[nb], sems[nb])
            pltpu.make_async_copy(u_sh.at[sidx_v.at[j]], rows_v.at[b], sems[b]).wait()
            pltpu.sync_copy(rows_v.at[b], acc_sh.at[didx_v.at[j]], add=True)
        return carry

    lax.fori_loop(0, CPW // 2, body, 0)
    pltpu.make_async_copy(u_sh.at[sidx_v.at[CPW]], rows_v.at[0], sems[0]).wait()

    plsc.subcore_barrier()
    pltpu.sync_copy(
        acc_sh.at[pl.ds(s * RPS, RPS)], out_hbm.at[c, pl.ds(s * RPS, RPS)]
    )


def _mm_body(x_ref, w_ref, o_ref):
    o_ref[...] = jnp.dot(
        x_ref[...].astype(jnp.bfloat16), w_ref[...].astype(jnp.bfloat16),
        preferred_element_type=jnp.float32,
    )


def _mm(x_pad, W1):
    return pl.pallas_call(
        _mm_body,
        grid=(GRID,),
        in_specs=[
            pl.BlockSpec((BM, F_IN), lambda i: (i, 0)),
            pl.BlockSpec((F_IN, H), lambda i: (0, 0)),
        ],
        out_specs=pl.BlockSpec((BM, H), lambda i: (i, 0)),
        out_shape=jax.ShapeDtypeStruct((N_PAD, H), jnp.float32),
    )(x_pad, W1)


def _scale_body(t_ref, d0_ref, d1_ref, u_ref, s_ref):
    i = pl.program_id(0)
    deg = d0_ref[...][:, 0:1] + d1_ref[...][:, 0:1]
    row = lax.broadcasted_iota(jnp.int32, (BM, 1), 0) + i * BM
    sc = jnp.where(row < N, 1.0 / jnp.sqrt(deg + 1.0), 0.0)
    s64 = jnp.broadcast_to(sc, (BM, H))
    s_ref[...] = s64
    u_ref[...] = t_ref[...] * s64


def _scale(t1, d0, d1):
    return pl.pallas_call(
        _scale_body,
        grid=(GRID,),
        in_specs=[
            pl.BlockSpec((BM, H), lambda i: (i, 0)),
            pl.BlockSpec((BM, 16), lambda i: (i, 0)),
            pl.BlockSpec((BM, 16), lambda i: (i, 0)),
        ],
        out_specs=[
            pl.BlockSpec((BM, H), lambda i: (i, 0)),
            pl.BlockSpec((BM, H), lambda i: (i, 0)),
        ],
        out_shape=[
            jax.ShapeDtypeStruct((N_PAD, H), jnp.float32),
            jax.ShapeDtypeStruct((N_PAD, H), jnp.float32),
        ],
    )(t1, d0, d1)


def _layer2_body(a0_ref, a1_ref, u1_ref, s_ref, w_ref, b_ref, o_ref):
    s = s_ref[...]
    h = jnp.maximum(s * (a0_ref[...] + a1_ref[...] + u1_ref[...]) + b_ref[...], 0.0)
    o_ref[...] = jnp.dot(
        h.astype(jnp.bfloat16), w_ref[...].astype(jnp.bfloat16),
        preferred_element_type=jnp.float32,
    ) * s


def _layer2(a0, a1, u1, s64, W2, b1_row):
    return pl.pallas_call(
        _layer2_body,
        grid=(GRID,),
        in_specs=[
            pl.BlockSpec((BM, H), lambda i: (i, 0)),
            pl.BlockSpec((BM, H), lambda i: (i, 0)),
            pl.BlockSpec((BM, H), lambda i: (i, 0)),
            pl.BlockSpec((BM, H), lambda i: (i, 0)),
            pl.BlockSpec((H, H), lambda i: (0, 0)),
            pl.BlockSpec((1, H), lambda i: (0, 0)),
        ],
        out_specs=pl.BlockSpec((BM, H), lambda i: (i, 0)),
        out_shape=jax.ShapeDtypeStruct((N_PAD, H), jnp.float32),
    )(a0, a1, u1, s64, W2, b1_row)


def _final_body(
    a0_ref, a1_ref, u2_ref, s_ref, b_ref, wfc_ref, bfc_ref, batch_ref,
    ia_ref, ib_ref, util_ref, pairs_ref, sum_sc, cnt_sc
):
    i = pl.program_id(0)

    @pl.when(i == 0)
    def _():
        sum_sc[...] = jnp.zeros_like(sum_sc)
        cnt_sc[...] = jnp.zeros_like(cnt_sc)

    s = s_ref[...]
    h = jnp.maximum(s * (a0_ref[...] + a1_ref[...] + u2_ref[...]) + b_ref[...], 0.0)
    z = jnp.dot(
        h.astype(jnp.bfloat16), wfc_ref[...].astype(jnp.bfloat16),
        preferred_element_type=jnp.float32,
    ) + bfc_ref[...]
    b = batch_ref[...]
    oh = (b == lax.broadcasted_iota(jnp.int32, (BM, G), 1)).astype(jnp.float32)
    dn = (((0,), (0,)), ((), ()))
    sum_sc[...] += lax.dot_general(oh, z, dn, preferred_element_type=jnp.float32, precision=lax.Precision.HIGHEST)
    cnt_sc[...] += lax.dot_general(
        oh, jnp.ones((BM, 1), jnp.float32), dn, preferred_element_type=jnp.float32,
        precision=lax.Precision.HIGHEST
    )

    @pl.when(i == GRID - 1)
    def _():
        util = sum_sc[...] / jnp.clip(cnt_sc[...], 1.0, None)
        util_ref[...] = util
        iot = lax.broadcasted_iota(jnp.int32, (P, G), 1)
        d = (ib_ref[...] == iot).astype(jnp.float32) - (
            ia_ref[...] == iot
        ).astype(jnp.float32)
        pairs_ref[...] = lax.dot_general(
            d, util, (((1,), (0,)), ((), ())), preferred_element_type=jnp.float32,
            precision=lax.Precision.HIGHEST
        )


def _final(a0, a1, u2, s64, b2_row, Wfc, bfc_row, batch_pad, ia, ib):
    return pl.pallas_call(
        _final_body,
        grid=(GRID,),
        in_specs=[
            pl.BlockSpec((BM, H), lambda i: (i, 0)),
            pl.BlockSpec((BM, H), lambda i: (i, 0)),
            pl.BlockSpec((BM, H), lambda i: (i, 0)),
            pl.BlockSpec((BM, H), lambda i: (i, 0)),
            pl.BlockSpec((1, H), lambda i: (0, 0)),
            pl.BlockSpec((H, 1), lambda i: (0, 0)),
            pl.BlockSpec((1, 1), lambda i: (0, 0)),
            pl.BlockSpec((BM, 1), lambda i: (i, 0)),
            pl.BlockSpec((P, 1), lambda i: (0, 0)),
            pl.BlockSpec((P, 1), lambda i: (0, 0)),
        ],
        out_specs=[
            pl.BlockSpec((G, 1), lambda i: (0, 0)),
            pl.BlockSpec((P, 1), lambda i: (0, 0)),
        ],
        out_shape=[
            jax.ShapeDtypeStruct((G, 1), jnp.float32),
            jax.ShapeDtypeStruct((P, 1), jnp.float32),
        ],
        scratch_shapes=[
            pltpu.VMEM((G, 1), jnp.float32),
            pltpu.VMEM((G, 1), jnp.float32),
        ],
    )(a0, a1, u2, s64, b2_row, Wfc, bfc_row, batch_pad, ia, ib)


def kernel(x, edge_index, batch, idx_a, idx_b, W1, b1, W2, b2, Wfc, bfc):
    x_pad = jnp.zeros((N_PAD, F_IN), jnp.float32).at[:N, :].set(x)
    pad_idx = jnp.full((E_PAD + CH - E,), DUMMY, jnp.int32)
    src = jnp.concatenate([edge_index[0], pad_idx]).reshape(-1, CH)
    dst = jnp.concatenate([edge_index[1], pad_idx]).reshape(-1, CH)
    batch_pad = jnp.concatenate(
        [batch, jnp.full((N_PAD - N,), G, jnp.int32)]
    ).reshape(N_PAD, 1)
    ia = idx_a.reshape(P, 1)
    ib = idx_b.reshape(P, 1)

    deg2 = _sc_degree(dst)
    t1 = _mm(x_pad, W1)
    u1, s64 = _scale(t1, deg2[0], deg2[1])
    acc1 = _sc_edge_scatter(u1, src, dst)
    u2 = _layer2(acc1[0], acc1[1], u1, s64, W2, b1.reshape(1, H))
    acc2 = _sc_edge_scatter(u2, src, dst)
    util, pairs = _final(
        acc2[0], acc2[1], u2, s64, b2.reshape(1, H), Wfc, bfc.reshape(1, 1),
        batch_pad, ia, ib
    )
    return pairs.reshape(P), util


# speedup vs baseline: 26.2960x; 1.0323x over previous
import functools

import jax
import jax.numpy as jnp
from jax import lax
from jax.experimental import pallas as pl
from jax.experimental.pallas import tpu as pltpu
from jax.experimental.pallas import tpu_sc as plsc

N = 10000
E = 320000
F_IN = 128
H = 64
G = 512
P = 1024

NC = 2
NS = 16
CH = 128
CPW = 80
E_PAD = NC * NS * CPW * CH
N_PAD = 10240
RPS = N_PAD // NS
DUMMY = N

BM = 256
GRID = N_PAD // BM

@functools.lru_cache(maxsize=None)
def _mesh():
    return plsc.VectorSubcoreMesh(
        core_axis_name="c", subcore_axis_name="s", num_cores=NC, num_subcores=NS
    )


def _fill(buf, value, ncols):
    v = jnp.full((16,), value, jnp.float32)

    def body(i, carry):
        for j in range(ncols):
            buf[i, pl.ds(j * 16, 16)] = v
        return carry

    lax.fori_loop(0, CH, body, 0)


@functools.lru_cache(maxsize=None)
def _build_sc_degree():
    return pl.kernel(
        _sc_degree_body,
        out_type=jax.ShapeDtypeStruct((NC, N_PAD, 16), jnp.float32),
        mesh=_mesh(),
        scratch_types=[
            pltpu.VMEM((CPW, CH), jnp.int32),
            pltpu.VMEM((CH, 16), jnp.float32),
            pltpu.VMEM_SHARED((N_PAD, 16), jnp.float32),
        ],
        compiler_params=pltpu.CompilerParams(use_tc_tiling_on_sc=False),
    )


def _sc_degree(dst):
    return _build_sc_degree()(dst)


def _sc_degree_body(dst_hbm, out_hbm, didx_v, ones_v, deg_sh):
    c = lax.axis_index("c")
    s = lax.axis_index("s")
    wid = c * NS + s

    _fill(ones_v, 0.0, 1)

    def zbody(j, carry):
        pltpu.sync_copy(ones_v, deg_sh.at[pl.ds(s * RPS + j * CH, CH)])
        return carry

    lax.fori_loop(0, RPS // CH, zbody, 0)
    plsc.subcore_barrier()

    _fill(ones_v, 1.0, 1)
    pltpu.sync_copy(dst_hbm.at[pl.ds(wid * CPW, CPW)], didx_v)

    def body(j, carry):
        pltpu.sync_copy(ones_v, deg_sh.at[didx_v.at[j]], add=True)
        return carry

    lax.fori_loop(0, CPW, body, 0)
    plsc.subcore_barrier()
    pltpu.sync_copy(
        deg_sh.at[pl.ds(s * RPS, RPS)], out_hbm.at[c, pl.ds(s * RPS, RPS)]
    )


@functools.lru_cache(maxsize=None)
def _build_sc_edge_scatter():
    return pl.kernel(
        _sc_edge_scatter_body,
        out_type=jax.ShapeDtypeStruct((NC, N_PAD, H), jnp.float32),
        mesh=_mesh(),
        scratch_types=[
            pltpu.VMEM((CPW + 1, CH), jnp.int32),
            pltpu.VMEM((CPW, CH), jnp.int32),
            pltpu.VMEM((2, CH, H), jnp.float32),
            pltpu.VMEM_SHARED((N_PAD, H), jnp.float32),
            pltpu.VMEM_SHARED((N_PAD, H), jnp.float32),
            pltpu.SemaphoreType.DMA,
            pltpu.SemaphoreType.DMA,
        ],
        compiler_params=pltpu.CompilerParams(use_tc_tiling_on_sc=False),
    )


def _sc_edge_scatter(u, src, dst):
    return _build_sc_edge_scatter()(u, src, dst)


def _sc_edge_scatter_body(u_hbm, src_hbm, dst_hbm, out_hbm, sidx_v, didx_v, rows_v, acc_sh, u_sh, sem0, sem1):
    c = lax.axis_index("c")
    s = lax.axis_index("s")
    wid = c * NS + s
    sems = (sem0, sem1)

    pltpu.sync_copy(u_hbm.at[pl.ds(s * RPS, RPS)], u_sh.at[pl.ds(s * RPS, RPS)])

    _fill(rows_v.at[0], 0.0, H // 16)

    def zbody(j, carry):
        pltpu.sync_copy(rows_v.at[0], acc_sh.at[pl.ds(s * RPS + j * CH, CH)])
        return carry

    lax.fori_loop(0, RPS // CH, zbody, 0)
    plsc.subcore_barrier()

    base = wid * CPW
    pltpu.sync_copy(src_hbm.at[pl.ds(base, CPW + 1)], sidx_v)
    pltpu.sync_copy(dst_hbm.at[pl.ds(base, CPW)], didx_v)
    pltpu.async_copy(u_sh.at[sidx_v.at[0]], rows_v.at[0], sems[0])

    def body(i, carry):
        for b in range(2):
            j = 2 * i + b
            nb = 1 - b
            pltpu.async_copy(u_sh.at[sidx_v.at[j + 1]], rows_v.at[nb], sems[nb])
            pltpu.make_async_copy(u_sh.at[sidx_v.at[j]], rows_v.at[b], sems[b]).wait()
            pltpu.sync_copy(rows_v.at[b], acc_sh.at[didx_v.at[j]], add=True)
        return carry

    lax.fori_loop(0, CPW // 2, body, 0)
    pltpu.make_async_copy(u_sh.at[sidx_v.at[CPW]], rows_v.at[0], sems[0]).wait()

    plsc.subcore_barrier()
    pltpu.sync_copy(
        acc_sh.at[pl.ds(s * RPS, RPS)], out_hbm.at[c, pl.ds(s * RPS, RPS)]
    )


def _scale_body(x_ref, w_ref, d0_ref, d1_ref, u_ref, s_ref):
    i = pl.program_id(0)
    t = jnp.dot(
        x_ref[...].astype(jnp.bfloat16), w_ref[...].astype(jnp.bfloat16),
        preferred_element_type=jnp.float32,
    )
    deg = d0_ref[...][:, 0:1] + d1_ref[...][:, 0:1]
    row = lax.broadcasted_iota(jnp.int32, (BM, 1), 0) + i * BM
    sc = jnp.where(row < N, 1.0 / jnp.sqrt(deg + 1.0), 0.0)
    s64 = jnp.broadcast_to(sc, (BM, H))
    s_ref[...] = s64
    u_ref[...] = t * s64


def _scale(x_pad, W1, d0, d1):
    return pl.pallas_call(
        _scale_body,
        grid=(GRID,),
        in_specs=[
            pl.BlockSpec((BM, F_IN), lambda i: (i, 0)),
            pl.BlockSpec((F_IN, H), lambda i: (0, 0)),
            pl.BlockSpec((BM, 16), lambda i: (i, 0)),
            pl.BlockSpec((BM, 16), lambda i: (i, 0)),
        ],
        out_specs=[
            pl.BlockSpec((BM, H), lambda i: (i, 0)),
            pl.BlockSpec((BM, H), lambda i: (i, 0)),
        ],
        out_shape=[
            jax.ShapeDtypeStruct((N_PAD, H), jnp.float32),
            jax.ShapeDtypeStruct((N_PAD, H), jnp.float32),
        ],
    )(x_pad, W1, d0, d1)


def _layer2_body(a0_ref, a1_ref, u1_ref, s_ref, w_ref, b_ref, o_ref):
    s = s_ref[...]
    h = jnp.maximum(s * (a0_ref[...] + a1_ref[...] + u1_ref[...]) + b_ref[...], 0.0)
    o_ref[...] = jnp.dot(
        h.astype(jnp.bfloat16), w_ref[...].astype(jnp.bfloat16),
        preferred_element_type=jnp.float32,
    ) * s


def _layer2(a0, a1, u1, s64, W2, b1_row):
    return pl.pallas_call(
        _layer2_body,
        grid=(GRID,),
        in_specs=[
            pl.BlockSpec((BM, H), lambda i: (i, 0)),
            pl.BlockSpec((BM, H), lambda i: (i, 0)),
            pl.BlockSpec((BM, H), lambda i: (i, 0)),
            pl.BlockSpec((BM, H), lambda i: (i, 0)),
            pl.BlockSpec((H, H), lambda i: (0, 0)),
            pl.BlockSpec((1, H), lambda i: (0, 0)),
        ],
        out_specs=pl.BlockSpec((BM, H), lambda i: (i, 0)),
        out_shape=jax.ShapeDtypeStruct((N_PAD, H), jnp.float32),
    )(a0, a1, u1, s64, W2, b1_row)


def _final_body(
    a0_ref, a1_ref, u2_ref, s_ref, b_ref, wfc_ref, bfc_ref, batch_ref,
    ia_ref, ib_ref, util_ref, pairs_ref, sum_sc, cnt_sc
):
    i = pl.program_id(0)

    @pl.when(i == 0)
    def _():
        sum_sc[...] = jnp.zeros_like(sum_sc)
        cnt_sc[...] = jnp.zeros_like(cnt_sc)

    s = s_ref[...]
    h = jnp.maximum(s * (a0_ref[...] + a1_ref[...] + u2_ref[...]) + b_ref[...], 0.0)
    z = jnp.dot(
        h.astype(jnp.bfloat16), wfc_ref[...].astype(jnp.bfloat16),
        preferred_element_type=jnp.float32,
    ) + bfc_ref[...]
    b = batch_ref[...]
    oh = (b == lax.broadcasted_iota(jnp.int32, (BM, G), 1)).astype(jnp.float32)
    dn = (((0,), (0,)), ((), ()))
    sum_sc[...] += lax.dot_general(oh, z, dn, preferred_element_type=jnp.float32, precision=lax.Precision.HIGHEST)
    cnt_sc[...] += lax.dot_general(
        oh, jnp.ones((BM, 1), jnp.float32), dn, preferred_element_type=jnp.float32,
        precision=lax.Precision.HIGHEST
    )

    @pl.when(i == GRID - 1)
    def _():
        util = sum_sc[...] / jnp.clip(cnt_sc[...], 1.0, None)
        util_ref[...] = util
        iot = lax.broadcasted_iota(jnp.int32, (P, G), 1)
        d = (ib_ref[...] == iot).astype(jnp.float32) - (
            ia_ref[...] == iot
        ).astype(jnp.float32)
        pairs_ref[...] = lax.dot_general(
            d, util, (((1,), (0,)), ((), ())), preferred_element_type=jnp.float32,
            precision=lax.Precision.HIGHEST
        )


def _final(a0, a1, u2, s64, b2_row, Wfc, bfc_row, batch_pad, ia, ib):
    return pl.pallas_call(
        _final_body,
        grid=(GRID,),
        in_specs=[
            pl.BlockSpec((BM, H), lambda i: (i, 0)),
            pl.BlockSpec((BM, H), lambda i: (i, 0)),
            pl.BlockSpec((BM, H), lambda i: (i, 0)),
            pl.BlockSpec((BM, H), lambda i: (i, 0)),
            pl.BlockSpec((1, H), lambda i: (0, 0)),
            pl.BlockSpec((H, 1), lambda i: (0, 0)),
            pl.BlockSpec((1, 1), lambda i: (0, 0)),
            pl.BlockSpec((BM, 1), lambda i: (i, 0)),
            pl.BlockSpec((P, 1), lambda i: (0, 0)),
            pl.BlockSpec((P, 1), lambda i: (0, 0)),
        ],
        out_specs=[
            pl.BlockSpec((G, 1), lambda i: (0, 0)),
            pl.BlockSpec((P, 1), lambda i: (0, 0)),
        ],
        out_shape=[
            jax.ShapeDtypeStruct((G, 1), jnp.float32),
            jax.ShapeDtypeStruct((P, 1), jnp.float32),
        ],
        scratch_shapes=[
            pltpu.VMEM((G, 1), jnp.float32),
            pltpu.VMEM((G, 1), jnp.float32),
        ],
    )(a0, a1, u2, s64, b2_row, Wfc, bfc_row, batch_pad, ia, ib)


def kernel(x, edge_index, batch, idx_a, idx_b, W1, b1, W2, b2, Wfc, bfc):
    x_pad = jnp.zeros((N_PAD, F_IN), jnp.float32).at[:N, :].set(x)
    pad_idx = jnp.full((E_PAD + CH - E,), DUMMY, jnp.int32)
    src = jnp.concatenate([edge_index[0], pad_idx]).reshape(-1, CH)
    dst = jnp.concatenate([edge_index[1], pad_idx]).reshape(-1, CH)
    batch_pad = jnp.concatenate(
        [batch, jnp.full((N_PAD - N,), G, jnp.int32)]
    ).reshape(N_PAD, 1)
    ia = idx_a.reshape(P, 1)
    ib = idx_b.reshape(P, 1)

    deg2 = _sc_degree(dst)
    u1, s64 = _scale(x_pad, W1, deg2[0], deg2[1])
    acc1 = _sc_edge_scatter(u1, src, dst)
    u2 = _layer2(acc1[0], acc1[1], u1, s64, W2, b1.reshape(1, H))
    acc2 = _sc_edge_scatter(u2, src, dst)
    util, pairs = _final(
        acc2[0], acc2[1], u2, s64, b2.reshape(1, H), Wfc, bfc.reshape(1, 1),
        batch_pad, ia, ib
    )
    return pairs.reshape(P), util
